# Initial kernel scaffold; baseline (speedup 1.0000x reference)
import jax
import jax.numpy as jnp
from jax.experimental import pallas as pl


def kernel(x, edge_index, W1_l, b1_l, W1_r, W2_l, b2_l, W2_r, Wc, bc):
    raise NotImplementedError("write your pallas kernel here")


# speedup vs baseline: 4.0663x; 4.0663x over previous
import jax
import jax.numpy as jnp
from jax import lax
from jax.experimental import pallas as pl
from jax.experimental.pallas import tpu as pltpu
from jax.experimental.pallas import tpu_sc as plsc

N = 10000
D = 128
E = 320000

NC = 2
NS = 16
NW = NC * NS

C = 128
G = 80
G2 = G // 2
EPT = G * C
E_PAD = NW * EPT

RPT = 640
N_PAD = NS * RPT

_mesh = plsc.VectorSubcoreMesh(core_axis_name="c", subcore_axis_name="s")


def _agg_body(feat, srcr, dstr, zrows, part0, part1,
              agg_sh, src_v, dst_v, rows0, rows1, sem0, sem1):
  cid = lax.axis_index("c")
  sid = lax.axis_index("s")
  wid = sid * NC + cid
  row0 = sid * RPT

  pltpu.sync_copy(zrows, agg_sh.at[pl.ds(row0, RPT)])
  plsc.subcore_barrier()

  def start(g, buf, sem):
    pltpu.async_copy(feat.at[src_v.at[g]], buf, sem)

  def finish(g, buf, sem):
    pltpu.make_async_copy(feat.at[src_v.at[g]], buf, sem).wait()

  def accum(g, buf):
    pltpu.sync_copy(buf, agg_sh.at[dst_v.at[g]], add=True)

  def run_slab(h):
    pltpu.sync_copy(srcr.at[wid, pl.ds(h * G2, G2)], src_v)
    pltpu.sync_copy(dstr.at[wid, pl.ds(h * G2, G2)], dst_v)

    start(0, rows0, sem0)
    start(1, rows1, sem1)

    @pl.loop(0, G2, step=2)
    def _(g):
      finish(g, rows0, sem0)
      accum(g, rows0)

      @pl.when(g + 2 < G2)
      def _():
        start(g + 2, rows0, sem0)

      finish(g + 1, rows1, sem1)
      accum(g + 1, rows1)

      @pl.when(g + 3 < G2)
      def _():
        start(g + 3, rows1, sem1)

  run_slab(0)
  run_slab(1)

  plsc.subcore_barrier()

  @pl.when(cid == 0)
  def _():
    pltpu.sync_copy(agg_sh.at[pl.ds(row0, RPT)], part0.at[pl.ds(row0, RPT)])

  @pl.when(cid == 1)
  def _():
    pltpu.sync_copy(agg_sh.at[pl.ds(row0, RPT)], part1.at[pl.ds(row0, RPT)])


_agg_only = pl.kernel(
    _agg_body,
    out_type=[jax.ShapeDtypeStruct((N_PAD, D), jnp.float32),
              jax.ShapeDtypeStruct((N_PAD, D), jnp.float32)],
    mesh=_mesh,
    scratch_types=[
        pltpu.VMEM_SHARED((N_PAD, D), jnp.float32),
        pltpu.VMEM((G2, C), jnp.int32),
        pltpu.VMEM((G2, C), jnp.int32),
        pltpu.VMEM((C, D), jnp.float32),
        pltpu.VMEM((C, D), jnp.float32),
        pltpu.SemaphoreType.DMA,
        pltpu.SemaphoreType.DMA,
    ],
)


def _deg_body(dstr, zdeg, ones1, deg0, deg1, deg_sh, dst_v, ones_v):
  cid = lax.axis_index("c")
  sid = lax.axis_index("s")
  wid = sid * NC + cid
  row0 = sid * RPT

  pltpu.sync_copy(zdeg, deg_sh.at[pl.ds(row0, RPT)])
  pltpu.sync_copy(ones1, ones_v)
  pltpu.sync_copy(dstr.at[wid], dst_v)

  plsc.subcore_barrier()

  @pl.loop(0, G)
  def _(g):
    pltpu.sync_copy(ones_v, deg_sh.at[dst_v.at[g]], add=True)

  plsc.subcore_barrier()

  @pl.when(cid == 0)
  def _():
    pltpu.sync_copy(deg_sh.at[pl.ds(row0, RPT)], deg0.at[pl.ds(row0, RPT)])

  @pl.when(cid == 1)
  def _():
    pltpu.sync_copy(deg_sh.at[pl.ds(row0, RPT)], deg1.at[pl.ds(row0, RPT)])


_deg_kernel = pl.kernel(
    _deg_body,
    out_type=[jax.ShapeDtypeStruct((N_PAD,), jnp.float32),
              jax.ShapeDtypeStruct((N_PAD,), jnp.float32)],
    mesh=_mesh,
    scratch_types=[
        pltpu.VMEM_SHARED((N_PAD,), jnp.float32),
        pltpu.VMEM((G, C), jnp.int32),
        pltpu.VMEM((C,), jnp.float32),
    ],
)


BN = 1000


def _layer_body(a0, a1, d0, d1, x, wl, bl, wr, out):
  deg = jnp.maximum(d0[...] + d1[...], 1.0)
  agg = (a0[...] + a1[...]) / deg
  h = lax.dot_general(agg, wl[...], (((1,), (1,)), ((), ())),
                      preferred_element_type=jnp.float32)
  h = h + bl[...] + lax.dot_general(x[...], wr[...], (((1,), (1,)), ((), ())),
                                    preferred_element_type=jnp.float32)
  out[...] = jnp.maximum(h, 0.0)


def _layer2_body(a0, a1, d0, d1, x, wl, bl, wr, wc, bc, out):
  deg = jnp.maximum(d0[...] + d1[...], 1.0)
  agg = (a0[...] + a1[...]) / deg
  h = lax.dot_general(agg, wl[...], (((1,), (1,)), ((), ())),
                      preferred_element_type=jnp.float32)
  h = h + bl[...] + lax.dot_general(x[...], wr[...], (((1,), (1,)), ((), ())),
                                    preferred_element_type=jnp.float32)
  h = jnp.maximum(h, 0.0)
  o = lax.dot_general(h, wc[...], (((1,), (0,)), ((), ())),
                      preferred_element_type=jnp.float32)
  out[...] = o + bc[...]


_row_spec = pl.BlockSpec((BN, D), lambda i: (i, 0))
_deg_spec = pl.BlockSpec((BN, 1), lambda i: (i, 0))
_w_spec = pl.BlockSpec((D, D), lambda i: (0, 0))
_b_spec = pl.BlockSpec((1, D), lambda i: (0, 0))

_layer_tc = pl.pallas_call(
    _layer_body,
    grid=(N // BN,),
    in_specs=[_row_spec, _row_spec, _deg_spec, _deg_spec, _row_spec,
              _w_spec, _b_spec, _w_spec],
    out_specs=_row_spec,
    out_shape=jax.ShapeDtypeStruct((N, D), jnp.float32),
)

_layer2_tc = pl.pallas_call(
    _layer2_body,
    grid=(N // BN,),
    in_specs=[_row_spec, _row_spec, _deg_spec, _deg_spec, _row_spec,
              _w_spec, _b_spec, _w_spec,
              pl.BlockSpec((D, 1), lambda i: (0, 0)),
              pl.BlockSpec((1, 1), lambda i: (0, 0))],
    out_specs=pl.BlockSpec((BN, 1), lambda i: (i, 0)),
    out_shape=jax.ShapeDtypeStruct((N, 1), jnp.float32),
)


def kernel(x, edge_index, W1_l, b1_l, W1_r, W2_l, b2_l, W2_r, Wc, bc):
  ei = edge_index.astype(jnp.int32)
  pad = E_PAD - E
  src = jnp.concatenate([ei[0], jnp.zeros((pad,), jnp.int32)])
  dst = jnp.concatenate([ei[1], jnp.full((pad,), N, jnp.int32)])
  srcr = src.reshape(NW, G, C)
  dstr = dst.reshape(NW, G, C)

  zrows = jnp.zeros((RPT, D), jnp.float32)
  zdeg = jnp.zeros((RPT,), jnp.float32)
  ones1 = jnp.ones((C,), jnp.float32)

  g0, g1 = _deg_kernel(dstr, zdeg, ones1)
  g0 = g0.reshape(N_PAD, 1)
  g1 = g1.reshape(N_PAD, 1)
  p0, p1 = _agg_only(x, srcr, dstr, zrows)
  h1 = _layer_tc(p0, p1, g0, g1, x,
                 W1_l, b1_l.reshape(1, D), W1_r)
  q0, q1 = _agg_only(h1, srcr, dstr, zrows)
  out = _layer2_tc(q0, q1, g0, g1, h1,
                   W2_l, b2_l.reshape(1, D), W2_r,
                   Wc.reshape(D, 1), bc.reshape(1, 1))
  return jnp.squeeze(out, axis=-1)
